# baseline (speedup 1.0000x reference)
import functools

import jax
import jax.numpy as jnp
import numpy as np
from jax import lax
from jax.experimental import pallas as pl
from jax.experimental.pallas import tpu as pltpu
from jax.experimental.pallas import tpu_sc as plsc

N_FIELDS = 26
VOCAB = 100000
EMB = 32
BATCH = 4096
NI = N_FIELDS + 1
DTOT = NI * EMB

NC, NS = 2, 16
NW = NC * NS
ROWS_TOT = BATCH * N_FIELDS
CHUNK = 128
NCHUNK = ROWS_TOT // (NW * CHUNK)


SAMP = BATCH // NW
RPW = SAMP * N_FIELDS


def _sc_gather(table, lsi):
    @functools.partial(
        pl.kernel,
        mesh=plsc.VectorSubcoreMesh(core_axis_name="c", subcore_axis_name="s"),
        out_type=jax.ShapeDtypeStruct((NW, RPW, EMB), jnp.float32),
        scratch_types=[
            pltpu.VMEM((N_FIELDS, SAMP), jnp.int32),
            pltpu.VMEM((RPW,), jnp.int32),
            pltpu.VMEM((RPW, EMB), jnp.float32),
            pltpu.SemaphoreType.DMA,
        ],
        compiler_params=pltpu.CompilerParams(
            use_tc_tiling_on_sc=False, needs_layout_passes=False),
    )
    def k(table_hbm, lsi_hbm, out_hbm, idx2_v, idxt_v, rows_v, sem):
        wid = lax.axis_index("s") * NC + lax.axis_index("c")
        pltpu.sync_copy(lsi_hbm.at[:, pl.ds(wid * SAMP, SAMP)], idx2_v)
        lane_sc = lax.iota(jnp.int32, 16) * N_FIELDS
        for kf in range(N_FIELDS):
            for g in range(SAMP // 16):
                v = idx2_v[kf, pl.ds(g * 16, 16)] + (kf * VOCAB)
                plsc.store_scatter(
                    idxt_v, [lane_sc + (g * 16 * N_FIELDS + kf)], v)
        half = NCHUNK // 2
        for lo in (0, half):
            cps = [
                pltpu.async_copy(
                    table_hbm.at[idxt_v.at[pl.ds(c * CHUNK, CHUNK)]],
                    rows_v.at[pl.ds(c * CHUNK, CHUNK)], sem)
                for c in range(lo, lo + half)
            ]
            for cp in cps:
                cp.wait()
        pltpu.sync_copy(rows_v, out_hbm.at[wid])

    return k(table, lsi)


def _tc_body(dx, emb, w0, b0, w1, b1, w2, b2, blocksum, wtop, tb0, wt1, tb1,
             wt2, tb2, out):
    f32 = jnp.float32
    bf16 = jnp.bfloat16
    x = jnp.maximum(jnp.dot(dx[...], w0[...], preferred_element_type=f32)
                    + b0[...], 0.0)
    x = jnp.maximum(jnp.dot(x.astype(bf16), w1[...],
                            preferred_element_type=f32) + b1[...], 0.0)
    x = jnp.maximum(jnp.dot(x.astype(bf16), w2[...],
                            preferred_element_type=f32) + b2[...], 0.0)
    xb = x.astype(bf16)
    t = jnp.concatenate([xb, emb[...]], axis=1)
    s_mat = blocksum[...]
    parts = [xb]
    for o in range(1, NI):
        s = o * EMB
        rolled = jnp.concatenate([t[:, DTOT - s:], t[:, :DTOT - s]], axis=1)
        zo = jnp.dot(t * rolled, s_mat, preferred_element_type=f32)
        parts.append(zo.astype(bf16))
    r = jnp.concatenate(parts, axis=1)
    h = jnp.maximum(jnp.dot(r, wtop[...], preferred_element_type=f32)
                    + tb0[...], 0.0)
    h = jnp.maximum(jnp.dot(h.astype(bf16), wt1[...],
                            preferred_element_type=f32) + tb1[...], 0.0)
    h = jnp.dot(h.astype(bf16), wt2[...], preferred_element_type=f32) \
        + tb2[...]
    out[...] = 1.0 / (1.0 + jnp.exp(-h))


def _expand_top_w(wt0):
    npairs = NI * (NI - 1) // 2
    sel = np.zeros((NI * N_FIELDS, npairs), dtype=np.float32)
    for o in range(1, NI):
        for i in range(NI):
            j = i - o
            if j >= 0:
                sel[(o - 1) * NI + i, i * (i - 1) // 2 + j] = 1.0
    wz = jnp.dot(jnp.asarray(sel), wt0[:, EMB:].T,
                 preferred_element_type=jnp.float32)
    return jnp.concatenate([wt0[:, :EMB].T, wz], axis=0)


def _tc_forward(dense_x, emb2d, bot_params, top_params, interpret=False):
    (w0, b0), (w1, b1), (w2, b2) = bot_params
    (t0, tb0), (t1, tb1), (t2, tb2) = top_params
    wtop = _expand_top_w(t0)
    bf16 = jnp.bfloat16
    blocksum = jnp.asarray(
        np.repeat(np.eye(NI, dtype=np.float32), EMB, axis=0)).astype(bf16)
    bt = 512
    grid = (BATCH // bt,)
    full = lambda a: pl.BlockSpec(a.shape, lambda i: (0,) * a.ndim)
    args = (
        dense_x, emb2d.astype(bf16),
        w0.T, b0.reshape(1, -1), w1.T.astype(bf16), b1.reshape(1, -1),
        w2.T.astype(bf16), b2.reshape(1, -1),
        blocksum, wtop.astype(bf16), tb0.reshape(1, -1),
        t1.T.astype(bf16), tb1.reshape(1, -1),
        t2.T.astype(bf16), tb2.reshape(1, -1),
    )
    in_specs = [
        pl.BlockSpec((bt, 13), lambda i: (i, 0)),
        pl.BlockSpec((bt, DTOT - EMB), lambda i: (i, 0)),
    ] + [full(a) for a in args[2:]]
    return pl.pallas_call(
        _tc_body,
        grid=grid,
        in_specs=in_specs,
        out_specs=pl.BlockSpec((bt, 1), lambda i: (i, 0)),
        out_shape=jax.ShapeDtypeStruct((BATCH, 1), jnp.float32),
        interpret=interpret,
    )(*args)


def kernel(dense_x, emb_tables, bot_params, top_params, lS_o, lS_i):
    del lS_o
    table = emb_tables.reshape(N_FIELDS * VOCAB, EMB)
    gathered = _sc_gather(table, lS_i)
    emb2d = gathered.reshape(BATCH, N_FIELDS * EMB)
    return _tc_forward(dense_x, emb2d, bot_params, top_params)


# speedup vs baseline: 1.0030x; 1.0030x over previous
import functools

import jax
import jax.numpy as jnp
import numpy as np
from jax import lax
from jax.experimental import pallas as pl
from jax.experimental.pallas import tpu as pltpu
from jax.experimental.pallas import tpu_sc as plsc

N_FIELDS = 26
VOCAB = 100000
EMB = 32
BATCH = 4096
NI = N_FIELDS + 1
DTOT = NI * EMB

NC, NS = 2, 16
NW = NC * NS
ROWS_TOT = BATCH * N_FIELDS
CHUNK = 128
NCHUNK = ROWS_TOT // (NW * CHUNK)


SAMP = BATCH // NW
RPW = SAMP * N_FIELDS
QPW = RPW // 4


def _sc_gather(table4, lsi_flat):
    @functools.partial(
        pl.kernel,
        mesh=plsc.VectorSubcoreMesh(core_axis_name="c", subcore_axis_name="s"),
        out_type=jax.ShapeDtypeStruct((NW, QPW, 128), jnp.float32),
        scratch_types=[
            pltpu.VMEM((N_FIELDS, SAMP), jnp.int32),
            pltpu.VMEM((RPW,), jnp.int32),
            pltpu.VMEM((RPW,), jnp.int32),
            pltpu.VMEM((CHUNK, 128), jnp.float32),
            pltpu.VMEM((CHUNK, 128), jnp.float32),
            pltpu.VMEM((CHUNK // 4, 128), jnp.float32),
            pltpu.VMEM((CHUNK // 4, 128), jnp.float32),
            pltpu.SemaphoreType.DMA,
            pltpu.SemaphoreType.DMA,
            pltpu.SemaphoreType.DMA,
            pltpu.SemaphoreType.DMA,
        ],
        compiler_params=pltpu.CompilerParams(
            use_tc_tiling_on_sc=True, needs_layout_passes=False),
    )
    def k(table_hbm, lsi_hbm, out_hbm, idx2_v, idxt_v, offs_v,
          slab_a, slab_b, ob_a, ob_b, sem_a, sem_b, wsem_a, wsem_b):
        wid = lax.axis_index("s") * NC + lax.axis_index("c")
        base = wid * SAMP
        cps = [
            pltpu.async_copy(
                lsi_hbm.at[pl.ds(kf * BATCH + base, SAMP)],
                idx2_v.at[kf], sem_a)
            for kf in range(N_FIELDS)
        ]
        for cp in cps:
            cp.wait()
        lane_sc = lax.iota(jnp.int32, 16) * N_FIELDS
        for kf in range(N_FIELDS):
            for g in range(SAMP // 16):
                gv = idx2_v[kf, pl.ds(g * 16, 16)] + (kf * VOCAB)
                tgt = lane_sc + (g * 16 * N_FIELDS + kf)
                plsc.store_scatter(idxt_v, [tgt], gv >> 2)
                plsc.store_scatter(offs_v, [tgt], (gv & 3) * EMB)
        slabs = (slab_a, slab_b)
        obufs = (ob_a, ob_b)
        sems = (sem_a, sem_b)
        wsems = (wsem_a, wsem_b)
        iota16 = lax.iota(jnp.int32, 16)

        def fire(c):
            return pltpu.async_copy(
                table_hbm.at[idxt_v.at[pl.ds(c * CHUNK, CHUNK)]],
                slabs[c % 2], sems[c % 2])

        cur = fire(0)
        wr = [None, None]
        for c in range(NCHUNK):
            cur.wait()
            if c + 1 < NCHUNK:
                nxt = fire(c + 1)
            if wr[c % 2] is not None:
                wr[c % 2].wait()
            slab = slabs[c % 2]
            obuf = obufs[c % 2]

            def ex_quad(i, _, c=c, slab=slab, obuf=obuf):
                for q in range(4):
                    r = i * 4 + q
                    j = c * CHUNK + r
                    off = plsc.load_gather(
                        offs_v, [jnp.full((16,), j, jnp.int32)])
                    col0 = off + iota16
                    row16 = jnp.full((16,), r, jnp.int32)
                    v0 = plsc.load_gather(slab, [row16, col0])
                    v1 = plsc.load_gather(slab, [row16, col0 + 16])
                    obuf[i, pl.ds(q * EMB, 16)] = v0
                    obuf[i, pl.ds(q * EMB + 16, 16)] = v1
                return 0

            lax.fori_loop(0, CHUNK // 4, ex_quad, 0)
            wr[c % 2] = pltpu.async_copy(
                obuf, out_hbm.at[wid, pl.ds(c * (CHUNK // 4), CHUNK // 4)],
                wsems[c % 2])
            if c + 1 < NCHUNK:
                cur = nxt
        for w in wr:
            if w is not None:
                w.wait()

    return k(table4, lsi_flat)


def _tc_body(dx, emb, w0, b0, w1, b1, w2, b2, blocksum, wtop, tb0, wt1, tb1,
             wt2, tb2, out):
    f32 = jnp.float32
    bf16 = jnp.bfloat16
    x = jnp.maximum(jnp.dot(dx[...], w0[...], preferred_element_type=f32)
                    + b0[...], 0.0)
    x = jnp.maximum(jnp.dot(x.astype(bf16), w1[...],
                            preferred_element_type=f32) + b1[...], 0.0)
    x = jnp.maximum(jnp.dot(x.astype(bf16), w2[...],
                            preferred_element_type=f32) + b2[...], 0.0)
    xb = x.astype(bf16)
    t = jnp.concatenate([xb, emb[...]], axis=1)
    s_mat = blocksum[...]
    parts = [xb]
    for o in range(1, NI):
        s = o * EMB
        rolled = jnp.concatenate([t[:, DTOT - s:], t[:, :DTOT - s]], axis=1)
        zo = jnp.dot(t * rolled, s_mat, preferred_element_type=f32)
        parts.append(zo.astype(bf16))
    r = jnp.concatenate(parts, axis=1)
    h = jnp.maximum(jnp.dot(r, wtop[...], preferred_element_type=f32)
                    + tb0[...], 0.0)
    h = jnp.maximum(jnp.dot(h.astype(bf16), wt1[...],
                            preferred_element_type=f32) + tb1[...], 0.0)
    h = jnp.dot(h.astype(bf16), wt2[...], preferred_element_type=f32) \
        + tb2[...]
    out[...] = 1.0 / (1.0 + jnp.exp(-h))


def _expand_top_w(wt0):
    npairs = NI * (NI - 1) // 2
    sel = np.zeros((NI * N_FIELDS, npairs), dtype=np.float32)
    for o in range(1, NI):
        for i in range(NI):
            j = i - o
            if j >= 0:
                sel[(o - 1) * NI + i, i * (i - 1) // 2 + j] = 1.0
    wz = jnp.dot(jnp.asarray(sel), wt0[:, EMB:].T,
                 preferred_element_type=jnp.float32)
    return jnp.concatenate([wt0[:, :EMB].T, wz], axis=0)


def _tc_forward(dense_x, emb2d, bot_params, top_params, interpret=False):
    (w0, b0), (w1, b1), (w2, b2) = bot_params
    (t0, tb0), (t1, tb1), (t2, tb2) = top_params
    wtop = _expand_top_w(t0)
    bf16 = jnp.bfloat16
    blocksum = jnp.asarray(
        np.repeat(np.eye(NI, dtype=np.float32), EMB, axis=0)).astype(bf16)
    bt = 512
    grid = (BATCH // bt,)
    full = lambda a: pl.BlockSpec(a.shape, lambda i: (0,) * a.ndim)
    args = (
        dense_x, emb2d.astype(bf16),
        w0.T, b0.reshape(1, -1), w1.T.astype(bf16), b1.reshape(1, -1),
        w2.T.astype(bf16), b2.reshape(1, -1),
        blocksum, wtop.astype(bf16), tb0.reshape(1, -1),
        t1.T.astype(bf16), tb1.reshape(1, -1),
        t2.T.astype(bf16), tb2.reshape(1, -1),
    )
    in_specs = [
        pl.BlockSpec((bt, 13), lambda i: (i, 0)),
        pl.BlockSpec((bt, DTOT - EMB), lambda i: (i, 0)),
    ] + [full(a) for a in args[2:]]
    return pl.pallas_call(
        _tc_body,
        grid=grid,
        in_specs=in_specs,
        out_specs=pl.BlockSpec((bt, 1), lambda i: (i, 0)),
        out_shape=jax.ShapeDtypeStruct((BATCH, 1), jnp.float32),
        interpret=interpret,
    )(*args)


def kernel(dense_x, emb_tables, bot_params, top_params, lS_o, lS_i):
    del lS_o
    table4 = emb_tables.reshape(N_FIELDS * VOCAB // 4, 4 * EMB)
    gathered = _sc_gather(table4, lS_i.reshape(-1))
    emb2d = gathered.reshape(BATCH, N_FIELDS * EMB)
    return _tc_forward(dense_x, emb2d, bot_params, top_params)
